# baseline (speedup 1.0000x reference)
import jax
import jax.numpy as jnp
from jax.experimental import pallas as pl
from jax.experimental.pallas import tpu as pltpu

K = 4
N = 4096
C = 64
CB = 256
NCH = N // CB
NBUF = 3
RB = 512
NRB = N // RB
THRESH = 0.41


def _row_count(s):
    row_any = jnp.max(s, axis=1, keepdims=True) > THRESH
    return jnp.sum(row_any.astype(jnp.float32), axis=0, keepdims=True)


def _prop_kernel(adj_hbm, x_ref, out_ref, buf_ref, adj_bf_ref, h_ref, s_ref,
                 sem):
    def cp(ch, slot):
        return pltpu.make_async_copy(
            adj_hbm.at[pl.ds(ch * CB, CB), :], buf_ref.at[slot], sem.at[slot]
        )

    for ch in range(NBUF):
        cp(ch, ch).start()

    s0 = jax.nn.sigmoid(x_ref[...])
    s_ref[0] = s0.astype(jnp.bfloat16)
    cnt = [None] * (K + 1)
    cnt[0] = _row_count(s0)

    xb = x_ref[...].astype(jnp.bfloat16)

    for ch in range(NCH):
        slot = ch % NBUF
        cp(ch, slot).wait()
        rows = pl.ds(ch * CB, CB)
        adj_bf_ref[rows, :] = buf_ref[slot].astype(jnp.bfloat16)
        if ch + NBUF < NCH:
            cp(ch + NBUF, slot).start()

    for k in range(1, K + 1):
        hb = xb if k == 1 else h_ref[k - 2]

        def hop_body(j, ck, k=k, hb=hb):
            crows = pl.ds(j * RB, RB)
            part = jnp.dot(
                adj_bf_ref[crows, :], hb, preferred_element_type=jnp.float32
            )
            if k < K:
                h_ref[k - 1, crows, :] = part.astype(jnp.bfloat16)
            s = jax.nn.sigmoid(part)
            s_ref[k, crows, :] = s.astype(jnp.bfloat16)
            return ck + _row_count(s)

        cnt[k] = jax.lax.fori_loop(
            0, NRB, hop_body, jnp.zeros((1, 1), jnp.float32)
        )

    maxc = cnt[0]
    for k in range(1, K + 1):
        maxc = jnp.maximum(maxc, cnt[k])

    acc = (cnt[0] / maxc) * s_ref[0].astype(jnp.float32)
    for k in range(1, K + 1):
        acc = acc + (cnt[k] / maxc) * s_ref[k].astype(jnp.float32)
    out_ref[...] = acc


@jax.jit
def kernel(x, adj):
    return pl.pallas_call(
        _prop_kernel,
        in_specs=[
            pl.BlockSpec(memory_space=pltpu.MemorySpace.HBM),
            pl.BlockSpec(memory_space=pltpu.MemorySpace.VMEM),
        ],
        out_specs=pl.BlockSpec(memory_space=pltpu.MemorySpace.VMEM),
        out_shape=jax.ShapeDtypeStruct((N, C), jnp.float32),
        scratch_shapes=[
            pltpu.VMEM((NBUF, CB, N), jnp.float32),
            pltpu.VMEM((N, N), jnp.bfloat16),
            pltpu.VMEM((K - 1, N, C), jnp.bfloat16),
            pltpu.VMEM((K + 1, N, C), jnp.bfloat16),
            pltpu.SemaphoreType.DMA((NBUF,)),
        ],
        compiler_params=pltpu.CompilerParams(
            vmem_limit_bytes=64 * 1024 * 1024,
        ),
    )(adj, x)


# speedup vs baseline: 1.1078x; 1.1078x over previous
import jax
import jax.numpy as jnp
from jax.experimental import pallas as pl
from jax.experimental.pallas import tpu as pltpu

K = 4
N = 4096
C = 64
CB = 256
NCH = N // CB
NBUF = 3
RB = 512
NRB = N // RB
THRESH = 0.41


def _row_count(s):
    row_any = jnp.max(s, axis=1, keepdims=True) > THRESH
    return jnp.sum(row_any.astype(jnp.float32), axis=0, keepdims=True)


def _prop_kernel(adj_hbm, x_ref, out_ref, buf_ref, adj_bf_ref, h_ref, s_ref,
                 sem):
    def cp(ch, slot):
        return pltpu.make_async_copy(
            adj_hbm.at[pl.ds(ch * CB, CB), :], buf_ref.at[slot], sem.at[slot]
        )

    for ch in range(NBUF):
        cp(ch, ch).start()

    s0 = jax.nn.sigmoid(x_ref[...])
    s_ref[0] = s0.astype(jnp.bfloat16)
    cnt = [None] * (K + 1)
    cnt[0] = _row_count(s0)

    xb = x_ref[...].astype(jnp.bfloat16)

    for ch in range(NCH):
        slot = ch % NBUF
        cp(ch, slot).wait()
        rows = pl.ds(ch * CB, CB)
        adj_bf_ref[rows, :] = buf_ref[slot].astype(jnp.bfloat16)
        if ch + NBUF < NCH:
            cp(ch + NBUF, slot).start()
        if ch % 2 == 1:
            drows = pl.ds((ch - 1) * CB, 2 * CB)
            h1 = jnp.dot(
                adj_bf_ref[drows, :], xb, preferred_element_type=jnp.float32
            )
            h_ref[0, drows, :] = h1.astype(jnp.bfloat16)

    cnt1 = jnp.zeros((1, 1), jnp.float32)
    for k in range(2, K + 1):
        hb = h_ref[k - 2]
        ck = jnp.zeros((1, 1), jnp.float32)
        for j in range(NRB):
            crows = pl.ds(j * RB, RB)
            part = jnp.dot(
                adj_bf_ref[crows, :], hb, preferred_element_type=jnp.float32
            )
            if k == 2:
                s1 = jax.nn.sigmoid(h_ref[0, crows, :].astype(jnp.float32))
                s_ref[1, crows, :] = s1.astype(jnp.bfloat16)
                cnt1 = cnt1 + _row_count(s1)
            if k < K:
                h_ref[k - 1, crows, :] = part.astype(jnp.bfloat16)
            s = jax.nn.sigmoid(part)
            s_ref[k, crows, :] = s.astype(jnp.bfloat16)
            ck = ck + _row_count(s)
        cnt[k] = ck
    cnt[1] = cnt1

    maxc = cnt[0]
    for k in range(1, K + 1):
        maxc = jnp.maximum(maxc, cnt[k])

    acc = (cnt[0] / maxc) * s_ref[0].astype(jnp.float32)
    for k in range(1, K + 1):
        acc = acc + (cnt[k] / maxc) * s_ref[k].astype(jnp.float32)
    out_ref[...] = acc


@jax.jit
def kernel(x, adj):
    return pl.pallas_call(
        _prop_kernel,
        in_specs=[
            pl.BlockSpec(memory_space=pltpu.MemorySpace.HBM),
            pl.BlockSpec(memory_space=pltpu.MemorySpace.VMEM),
        ],
        out_specs=pl.BlockSpec(memory_space=pltpu.MemorySpace.VMEM),
        out_shape=jax.ShapeDtypeStruct((N, C), jnp.float32),
        scratch_shapes=[
            pltpu.VMEM((NBUF, CB, N), jnp.float32),
            pltpu.VMEM((N, N), jnp.bfloat16),
            pltpu.VMEM((K - 1, N, C), jnp.bfloat16),
            pltpu.VMEM((K + 1, N, C), jnp.bfloat16),
            pltpu.SemaphoreType.DMA((NBUF,)),
        ],
        compiler_params=pltpu.CompilerParams(
            vmem_limit_bytes=64 * 1024 * 1024,
        ),
    )(adj, x)


# speedup vs baseline: 1.2469x; 1.1256x over previous
import jax
import jax.numpy as jnp
from jax.experimental import pallas as pl
from jax.experimental.pallas import tpu as pltpu

K = 4
N = 4096
C = 64
CB = 256
NCH = N // CB
NBUF = 3
RB = 512
NRB = N // RB
THRESH = 0.41


def _row_count(s):
    row_any = jnp.max(s, axis=1, keepdims=True) > THRESH
    return jnp.sum(row_any.astype(jnp.float32), axis=0, keepdims=True)


def _prop_kernel(adj_hbm, x_ref, out_ref, buf_ref, adj_bf_ref, h_ref, s_ref,
                 sem):
    def cp(ch, slot):
        return pltpu.make_async_copy(
            adj_hbm.at[pl.ds(ch * CB, CB), :], buf_ref.at[slot], sem.at[slot]
        )

    for ch in range(NBUF):
        cp(ch, ch).start()

    s0 = jax.nn.sigmoid(x_ref[...])
    s_ref[0] = s0.astype(jnp.bfloat16)
    cnt = [None] * (K + 1)
    cnt[0] = _row_count(s0)

    xb = x_ref[...].astype(jnp.bfloat16)

    cnt1 = jnp.zeros((1, 1), jnp.float32)
    for ch in range(NCH):
        slot = ch % NBUF
        cp(ch, slot).wait()
        rows = pl.ds(ch * CB, CB)
        blk_bf = buf_ref[slot].astype(jnp.bfloat16)
        adj_bf_ref[rows, :] = blk_bf
        h1 = jnp.dot(blk_bf, xb, preferred_element_type=jnp.float32)
        h_ref[0, rows, :] = h1.astype(jnp.bfloat16)
        s1 = jax.nn.sigmoid(h1)
        s_ref[1, rows, :] = s1.astype(jnp.bfloat16)
        cnt1 = cnt1 + _row_count(s1)
        if ch + NBUF < NCH:
            cp(ch + NBUF, slot).start()
    cnt[1] = cnt1

    for k in range(2, K + 1):
        hb = h_ref[k - 2]
        ck = jnp.zeros((1, 1), jnp.float32)
        for j in range(NRB):
            crows = pl.ds(j * RB, RB)
            part = jnp.dot(
                adj_bf_ref[crows, :], hb, preferred_element_type=jnp.float32
            )
            if k < K:
                h_ref[k - 1, crows, :] = part.astype(jnp.bfloat16)
            s = jax.nn.sigmoid(part)
            s_ref[k, crows, :] = s.astype(jnp.bfloat16)
            ck = ck + _row_count(s)
        cnt[k] = ck

    maxc = cnt[0]
    for k in range(1, K + 1):
        maxc = jnp.maximum(maxc, cnt[k])

    acc = (cnt[0] / maxc) * s_ref[0].astype(jnp.float32)
    for k in range(1, K + 1):
        acc = acc + (cnt[k] / maxc) * s_ref[k].astype(jnp.float32)
    out_ref[...] = acc


@jax.jit
def kernel(x, adj):
    return pl.pallas_call(
        _prop_kernel,
        in_specs=[
            pl.BlockSpec(memory_space=pltpu.MemorySpace.HBM),
            pl.BlockSpec(memory_space=pltpu.MemorySpace.VMEM),
        ],
        out_specs=pl.BlockSpec(memory_space=pltpu.MemorySpace.VMEM),
        out_shape=jax.ShapeDtypeStruct((N, C), jnp.float32),
        scratch_shapes=[
            pltpu.VMEM((NBUF, CB, N), jnp.float32),
            pltpu.VMEM((N, N), jnp.bfloat16),
            pltpu.VMEM((K - 1, N, C), jnp.bfloat16),
            pltpu.VMEM((K + 1, N, C), jnp.bfloat16),
            pltpu.SemaphoreType.DMA((NBUF,)),
        ],
        compiler_params=pltpu.CompilerParams(
            vmem_limit_bytes=64 * 1024 * 1024,
        ),
    )(adj, x)
